# Initial kernel scaffold; baseline (speedup 1.0000x reference)
import jax
import jax.numpy as jnp
from jax.experimental import pallas as pl


def kernel(clouds, params):
    raise NotImplementedError("write your pallas kernel here")


# speedup vs baseline: 10.1926x; 10.1926x over previous
import functools

import jax
import jax.numpy as jnp
from jax import lax
from jax.experimental import pallas as pl

_B = 4
_N = 8192
_STAGES = [(1024, 0.1, 32), (512, 0.2, 32), (256, 0.4, 16)]
_BN_INV = 1.0 / (1.0 + 1e-5) ** 0.5


def _fps_kernel(x_ref, o_ref, *, npoint, n, b):
    x = x_ref[0, :, :]
    y = x_ref[1, :, :]
    z = x_ref[2, :, :]
    iota = lax.broadcasted_iota(jnp.int32, (b, n), 1)
    lx = x[:, 0:1]
    ly = y[:, 0:1]
    lz = z[:, 0:1]
    o_ref[0, :, :] = jnp.concatenate([lx, ly, lz], axis=1)
    dists = jnp.full((b, n), 1e10, dtype=jnp.float32)

    def body(i, carry):
        dists, lx, ly, lz = carry
        d = ((x - lx) ** 2 + (y - ly) ** 2) + (z - lz) ** 2
        dists = jnp.minimum(dists, d)
        m = jnp.max(dists, axis=1, keepdims=True)
        sel = jnp.where(dists == m, iota, n)
        nxt = jnp.min(sel, axis=1, keepdims=True)
        pick = iota == nxt
        lx = jnp.sum(jnp.where(pick, x, 0.0), axis=1, keepdims=True)
        ly = jnp.sum(jnp.where(pick, y, 0.0), axis=1, keepdims=True)
        lz = jnp.sum(jnp.where(pick, z, 0.0), axis=1, keepdims=True)
        o_ref[pl.ds(i, 1), :, :] = jnp.concatenate([lx, ly, lz], axis=1)[None]
        return dists, lx, ly, lz

    lax.fori_loop(1, npoint, body, (dists, lx, ly, lz))


def _fps_call(xyz_t, npoint):
    _, b, n = xyz_t.shape
    return pl.pallas_call(
        functools.partial(_fps_kernel, npoint=npoint, n=n, b=b),
        out_shape=jax.ShapeDtypeStruct((npoint, b, 3), jnp.float32),
        interpret=False,
    )(xyz_t)


def _bq_kernel(x_ref, c_ref, o_ref, *, n, k, r2, sb):
    x = x_ref[0, 0:1, :]
    y = x_ref[0, 1:2, :]
    z = x_ref[0, 2:3, :]
    cx = c_ref[0, :, 0:1]
    cy = c_ref[0, :, 1:2]
    cz = c_ref[0, :, 2:3]
    d2 = ((cx - x) ** 2 + (cy - y) ** 2) + (cz - z) ** 2
    iota = lax.broadcasted_iota(jnp.int32, (sb, n), 1)
    scores = jnp.where(d2 <= r2, iota, n)
    cols = []
    for j in range(k):
        m = jnp.min(scores, axis=1, keepdims=True)
        cols.append(m)
        if j < k - 1:
            scores = jnp.where(scores == m, n, scores)
    idx = jnp.concatenate(cols, axis=1)
    first = idx[:, 0:1]
    first = jnp.where(first < n, first, 0)
    idx = jnp.where(idx < n, idx, first)
    bi = pl.program_id(0)
    o_ref[0, :, :] = idx + bi * n


def _bq_call(xyz_t, centers, radius, k, sb):
    b, _, n = xyz_t.shape
    s = centers.shape[1]
    r2 = float(radius * radius)
    return pl.pallas_call(
        functools.partial(_bq_kernel, n=n, k=k, r2=r2, sb=sb),
        grid=(b, s // sb),
        in_specs=[
            pl.BlockSpec((1, 3, n), lambda bi, j: (bi, 0, 0)),
            pl.BlockSpec((1, sb, 3), lambda bi, j: (bi, j, 0)),
        ],
        out_specs=pl.BlockSpec((1, sb, k), lambda bi, j: (bi, j, 0)),
        out_shape=jax.ShapeDtypeStruct((b, s, k), jnp.int32),
        interpret=False,
    )(xyz_t, centers)


def _mlp_kernel(g_ref, c_ref, *refs, k, cfeat, sb, nlayers):
    o_ref = refs[-1]
    wrefs = refs[:-1]
    gx = g_ref[:, 0:3]
    c = c_ref[:, :]
    xyz = gx.reshape(sb, k, 3) - c[:, None, :]
    x = xyz.reshape(sb * k, 3)
    if cfeat:
        x = jnp.concatenate([x, g_ref[:, 3:3 + cfeat]], axis=1)
    for li in range(nlayers):
        w = wrefs[2 * li][:, :]
        bb = wrefs[2 * li + 1][:, :]
        x = jnp.maximum(
            jnp.dot(x, w, preferred_element_type=jnp.float32) + bb, 0.0)
    cout = x.shape[1]
    o_ref[:, :] = jnp.max(x.reshape(sb, k, cout), axis=1)


def _mlp_call(gathered, centers_rows, layers, k, cfeat, sb):
    rows, d = gathered.shape
    bs = rows // k
    cout = layers[-1][0].shape[1]
    nlayers = len(layers)
    wargs = []
    in_specs = [
        pl.BlockSpec((sb * k, d), lambda g: (g, 0)),
        pl.BlockSpec((sb, 3), lambda g: (g, 0)),
    ]
    for wt, b2 in layers:
        wargs += [wt, b2]
        in_specs += [
            pl.BlockSpec(wt.shape, lambda g: (0, 0)),
            pl.BlockSpec(b2.shape, lambda g: (0, 0)),
        ]
    return pl.pallas_call(
        functools.partial(_mlp_kernel, k=k, cfeat=cfeat, sb=sb,
                          nlayers=nlayers),
        grid=(bs // sb,),
        in_specs=in_specs,
        out_specs=pl.BlockSpec((sb, cout), lambda g: (g, 0)),
        out_shape=jax.ShapeDtypeStruct((bs, cout), jnp.float32),
        interpret=False,
    )(gathered, centers_rows, *wargs)


def _gather_rows(table, idx_flat):
    return jnp.take(table, idx_flat, axis=0)


_SB_BQ = [256, 512, 256]
_SB_MLP = [512, 512, 256]
_PAD_D = [16, 48, 80]


def kernel(clouds, params):
    xyz = clouds[..., 0:3]
    feats = None
    for si, ((npoint, radius, nsample), stage_p) in enumerate(
            zip(_STAGES, params)):
        b, n, _ = xyz.shape
        xyz_t = jnp.transpose(xyz, (2, 0, 1))
        cent = _fps_call(xyz_t, npoint)
        new_xyz = jnp.transpose(cent, (1, 0, 2))
        idx = _bq_call(jnp.transpose(xyz, (0, 2, 1)), new_xyz, radius,
                       nsample, _SB_BQ[si])
        cfeat = 0 if feats is None else feats.shape[-1]
        base = xyz if feats is None else jnp.concatenate([xyz, feats], -1)
        d = _PAD_D[si]
        table = jnp.pad(base, ((0, 0), (0, 0), (0, d - 3 - cfeat)))
        table = table.reshape(b * n, d)
        gathered = _gather_rows(table, idx.reshape(-1))
        layers = []
        for lp in stage_p:
            s = lp["gamma"] * _BN_INV
            wt2 = (lp["W"] * s[:, None]).T
            b2 = (lp["b"] * s + lp["beta"])[None, :]
            layers.append((wt2, b2))
        fr = _mlp_call(gathered, new_xyz.reshape(b * npoint, 3), layers,
                       nsample, cfeat, _SB_MLP[si])
        feats = fr.reshape(b, npoint, fr.shape[-1])
        xyz = new_xyz
    return jnp.transpose(feats, (0, 2, 1))


# speedup vs baseline: 13.5199x; 1.3264x over previous
import functools

import jax
import jax.numpy as jnp
from jax import lax
from jax.experimental import pallas as pl
from jax.experimental.pallas import tpu as pltpu
from jax.experimental.pallas import tpu_sc as plsc

_B = 4
_N = 8192
_STAGES = [(1024, 0.1, 32), (512, 0.2, 32), (256, 0.4, 16)]
_BN_INV = 1.0 / (1.0 + 1e-5) ** 0.5


def _fps_kernel(x_ref, o_ref, *, npoint, n, b):
    x = x_ref[0, :, :]
    y = x_ref[1, :, :]
    z = x_ref[2, :, :]
    iota = lax.broadcasted_iota(jnp.int32, (b, n), 1)
    lx = x[:, 0:1]
    ly = y[:, 0:1]
    lz = z[:, 0:1]
    o_ref[0, :, :] = jnp.concatenate([lx, ly, lz], axis=1)
    dists = jnp.full((b, n), 1e10, dtype=jnp.float32)

    def body(i, carry):
        dists, lx, ly, lz = carry
        d = ((x - lx) ** 2 + (y - ly) ** 2) + (z - lz) ** 2
        dists = jnp.minimum(dists, d)
        m = jnp.max(dists, axis=1, keepdims=True)
        sel = jnp.where(dists == m, iota, n)
        nxt = jnp.min(sel, axis=1, keepdims=True)
        pick = iota == nxt
        lx = jnp.sum(jnp.where(pick, x, 0.0), axis=1, keepdims=True)
        ly = jnp.sum(jnp.where(pick, y, 0.0), axis=1, keepdims=True)
        lz = jnp.sum(jnp.where(pick, z, 0.0), axis=1, keepdims=True)
        o_ref[pl.ds(i, 1), :, :] = jnp.concatenate([lx, ly, lz], axis=1)[None]
        return dists, lx, ly, lz

    lax.fori_loop(1, npoint, body, (dists, lx, ly, lz))


def _fps_call(xyz_t, npoint):
    _, b, n = xyz_t.shape
    return pl.pallas_call(
        functools.partial(_fps_kernel, npoint=npoint, n=n, b=b),
        out_shape=jax.ShapeDtypeStruct((npoint, b, 3), jnp.float32),
        interpret=False,
    )(xyz_t)


def _bq_kernel(x_ref, c_ref, o_ref, *, n, k, r2, sb):
    x = x_ref[0, 0:1, :]
    y = x_ref[0, 1:2, :]
    z = x_ref[0, 2:3, :]
    cx = c_ref[0, :, 0:1]
    cy = c_ref[0, :, 1:2]
    cz = c_ref[0, :, 2:3]
    d2 = ((cx - x) ** 2 + (cy - y) ** 2) + (cz - z) ** 2
    iota = lax.broadcasted_iota(jnp.int32, (sb, n), 1)
    scores = jnp.where(d2 <= r2, iota, n)
    cols = []
    for j in range(k):
        m = jnp.min(scores, axis=1, keepdims=True)
        cols.append(m)
        if j < k - 1:
            scores = jnp.where(scores == m, n, scores)
    idx = jnp.concatenate(cols, axis=1)
    first = idx[:, 0:1]
    first = jnp.where(first < n, first, 0)
    idx = jnp.where(idx < n, idx, first)
    bi = pl.program_id(0)
    o_ref[0, :, :] = idx + bi * n


def _bq_call(xyz_t, centers, radius, k, sb):
    b, _, n = xyz_t.shape
    s = centers.shape[1]
    r2 = float(radius * radius)
    return pl.pallas_call(
        functools.partial(_bq_kernel, n=n, k=k, r2=r2, sb=sb),
        grid=(b, s // sb),
        in_specs=[
            pl.BlockSpec((1, 3, n), lambda bi, j: (bi, 0, 0)),
            pl.BlockSpec((1, sb, 3), lambda bi, j: (bi, j, 0)),
        ],
        out_specs=pl.BlockSpec((1, sb, k), lambda bi, j: (bi, j, 0)),
        out_shape=jax.ShapeDtypeStruct((b, s, k), jnp.int32),
        interpret=False,
    )(xyz_t, centers)


def _mlp_kernel(g_ref, c_ref, *refs, k, cfeat, sb, nlayers):
    o_ref = refs[-1]
    wrefs = refs[:-1]
    gx = g_ref[:, 0:3]
    c = c_ref[:, :]
    xyz = gx.reshape(sb, k, 3) - c[:, None, :]
    x = xyz.reshape(sb * k, 3)
    if cfeat:
        x = jnp.concatenate([x, g_ref[:, 3:3 + cfeat]], axis=1)
    for li in range(nlayers):
        w = wrefs[2 * li][:, :]
        bb = wrefs[2 * li + 1][:, :]
        x = jnp.maximum(
            jnp.dot(x, w, preferred_element_type=jnp.float32) + bb, 0.0)
    cout = x.shape[1]
    o_ref[:, :] = jnp.max(x.reshape(sb, k, cout), axis=1)


def _mlp_call(gathered, centers_rows, layers, k, cfeat, sb):
    rows, d = gathered.shape
    bs = rows // k
    cout = layers[-1][0].shape[1]
    nlayers = len(layers)
    wargs = []
    in_specs = [
        pl.BlockSpec((sb * k, d), lambda g: (g, 0)),
        pl.BlockSpec((sb, 3), lambda g: (g, 0)),
    ]
    for wt, b2 in layers:
        wargs += [wt, b2]
        in_specs += [
            pl.BlockSpec(wt.shape, lambda g: (0, 0)),
            pl.BlockSpec(b2.shape, lambda g: (0, 0)),
        ]
    return pl.pallas_call(
        functools.partial(_mlp_kernel, k=k, cfeat=cfeat, sb=sb,
                          nlayers=nlayers),
        grid=(bs // sb,),
        in_specs=in_specs,
        out_specs=pl.BlockSpec((sb, cout), lambda g: (g, 0)),
        out_shape=jax.ShapeDtypeStruct((bs, cout), jnp.float32),
        interpret=False,
    )(gathered, centers_rows, *wargs)


_SC_CHUNK = 128


def _sc_gather_body(table_hbm, idx_hbm, out_hbm, idx_v, rows_v, sem, *,
                    m_per, nc):
    wid = lax.axis_index("s") * nc + lax.axis_index("c")
    base = pl.multiple_of(wid * m_per, 8)
    pltpu.sync_copy(idx_hbm.at[pl.ds(base, m_per)], idx_v)

    def body(i, _):
        off = pl.multiple_of(i * _SC_CHUNK, 8)
        pltpu.async_copy(
            table_hbm.at[idx_v.at[pl.ds(off, _SC_CHUNK)]],
            rows_v.at[pl.ds(off, _SC_CHUNK)], sem).wait()
        return 0

    lax.fori_loop(0, m_per // _SC_CHUNK, body, 0)
    pltpu.sync_copy(rows_v, out_hbm.at[pl.ds(base, m_per)])


def _gather_rows(table, idx_flat):
    m = idx_flat.shape[0]
    d = table.shape[1]
    info = plsc.get_sparse_core_info()
    nw = info.num_cores * info.num_subcores
    m_per = m // nw
    mesh = plsc.VectorSubcoreMesh(core_axis_name="c", subcore_axis_name="s")
    f = pl.kernel(
        functools.partial(_sc_gather_body, m_per=m_per, nc=info.num_cores),
        mesh=mesh,
        out_type=jax.ShapeDtypeStruct((m, d), jnp.float32),
        scratch_types=[
            pltpu.VMEM((m_per,), jnp.int32),
            pltpu.VMEM((m_per, d), jnp.float32),
            pltpu.SemaphoreType.DMA,
        ],
        compiler_params=pltpu.CompilerParams(use_tc_tiling_on_sc=False),
    )
    return f(table, idx_flat)


_SB_BQ = [256, 512, 256]
_SB_MLP = [512, 512, 256]
_PAD_D = [16, 48, 80]


def kernel(clouds, params):
    xyz = clouds[..., 0:3]
    feats = None
    for si, ((npoint, radius, nsample), stage_p) in enumerate(
            zip(_STAGES, params)):
        b, n, _ = xyz.shape
        xyz_t = jnp.transpose(xyz, (2, 0, 1))
        cent = _fps_call(xyz_t, npoint)
        new_xyz = jnp.transpose(cent, (1, 0, 2))
        idx = _bq_call(jnp.transpose(xyz, (0, 2, 1)), new_xyz, radius,
                       nsample, _SB_BQ[si])
        cfeat = 0 if feats is None else feats.shape[-1]
        base = xyz if feats is None else jnp.concatenate([xyz, feats], -1)
        d = _PAD_D[si]
        table = jnp.pad(base, ((0, 0), (0, 0), (0, d - 3 - cfeat)))
        table = table.reshape(b * n, d)
        gathered = _gather_rows(table, idx.reshape(-1))
        layers = []
        for lp in stage_p:
            s = lp["gamma"] * _BN_INV
            wt2 = (lp["W"] * s[:, None]).T
            b2 = (lp["b"] * s + lp["beta"])[None, :]
            layers.append((wt2, b2))
        fr = _mlp_call(gathered, new_xyz.reshape(b * npoint, 3), layers,
                       nsample, cfeat, _SB_MLP[si])
        feats = fr.reshape(b, npoint, fr.shape[-1])
        xyz = new_xyz
    return jnp.transpose(feats, (0, 2, 1))


# speedup vs baseline: 20.3411x; 1.5045x over previous
import functools

import jax
import jax.numpy as jnp
from jax import lax
from jax.experimental import pallas as pl
from jax.experimental.pallas import tpu as pltpu
from jax.experimental.pallas import tpu_sc as plsc

_B = 4
_N = 8192
_STAGES = [(1024, 0.1, 32), (512, 0.2, 32), (256, 0.4, 16)]
_BN_INV = 1.0 / (1.0 + 1e-5) ** 0.5


def _fps_kernel(x_ref, o_ref, *, npoint, n, b):
    x = x_ref[0, :, :]
    y = x_ref[1, :, :]
    z = x_ref[2, :, :]
    iota = lax.broadcasted_iota(jnp.int32, (b, n), 1)
    lx = x[:, 0:1]
    ly = y[:, 0:1]
    lz = z[:, 0:1]
    o_ref[0, :, :] = jnp.concatenate([lx, ly, lz], axis=1)
    dists = jnp.full((b, n), 1e10, dtype=jnp.float32)

    def body(i, carry):
        dists, lx, ly, lz = carry
        d = ((x - lx) ** 2 + (y - ly) ** 2) + (z - lz) ** 2
        dists = jnp.minimum(dists, d)
        nxt = jnp.argmax(dists, axis=1, keepdims=True).astype(jnp.int32)
        pick = iota == nxt
        lx = jnp.sum(jnp.where(pick, x, 0.0), axis=1, keepdims=True)
        ly = jnp.sum(jnp.where(pick, y, 0.0), axis=1, keepdims=True)
        lz = jnp.sum(jnp.where(pick, z, 0.0), axis=1, keepdims=True)
        o_ref[pl.ds(i, 1), :, :] = jnp.concatenate([lx, ly, lz], axis=1)[None]
        return dists, lx, ly, lz

    lax.fori_loop(1, npoint, body, (dists, lx, ly, lz))


def _fps_call(xyz_t, npoint):
    _, b, n = xyz_t.shape
    return pl.pallas_call(
        functools.partial(_fps_kernel, npoint=npoint, n=n, b=b),
        out_shape=jax.ShapeDtypeStruct((npoint, b, 3), jnp.float32),
        interpret=False,
    )(xyz_t)


def _bq_kernel(x_ref, c_ref, o_ref, *, n, k, r2, sb):
    x = x_ref[0, 0:1, :]
    y = x_ref[0, 1:2, :]
    z = x_ref[0, 2:3, :]
    cx = c_ref[0, :, 0:1]
    cy = c_ref[0, :, 1:2]
    cz = c_ref[0, :, 2:3]
    d2 = ((cx - x) ** 2 + (cy - y) ** 2) + (cz - z) ** 2
    iota = lax.broadcasted_iota(jnp.int32, (sb, n), 1)
    scores = jnp.where(d2 <= r2, iota, n)

    def cond(st):
        j, _, _, more = st
        return jnp.logical_and(j < k, more)

    kiota = lax.broadcasted_iota(jnp.int32, (sb, k), 1)

    def body(st):
        j, scores, out, _ = st
        m = jnp.min(scores, axis=1, keepdims=True)
        out = jnp.where(kiota == j, m, out)
        scores = jnp.where(scores == m, n, scores)
        return j + 1, scores, out, jnp.min(m) < n

    init = (jnp.int32(0), scores,
            jnp.full((sb, k), n, dtype=jnp.int32), True)
    _, _, idx, _ = lax.while_loop(cond, body, init)
    first = idx[:, 0:1]
    first = jnp.where(first < n, first, 0)
    idx = jnp.where(idx < n, idx, first)
    bi = pl.program_id(0)
    o_ref[0, :, :] = idx + bi * n


def _bq_call(xyz_t, centers, radius, k, sb):
    b, _, n = xyz_t.shape
    s = centers.shape[1]
    r2 = float(radius * radius)
    return pl.pallas_call(
        functools.partial(_bq_kernel, n=n, k=k, r2=r2, sb=sb),
        grid=(b, s // sb),
        in_specs=[
            pl.BlockSpec((1, 3, n), lambda bi, j: (bi, 0, 0)),
            pl.BlockSpec((1, sb, 3), lambda bi, j: (bi, j, 0)),
        ],
        out_specs=pl.BlockSpec((1, sb, k), lambda bi, j: (bi, j, 0)),
        out_shape=jax.ShapeDtypeStruct((b, s, k), jnp.int32),
        interpret=False,
    )(xyz_t, centers)


def _mlp_kernel(g_ref, c_ref, *refs, k, cfeat, sb, nlayers):
    o_ref = refs[-1]
    wrefs = refs[:-1]
    gx = g_ref[:, 0:3]
    c = c_ref[:, :]
    xyz = gx.reshape(sb, k, 3) - c[:, None, :]
    x = xyz.reshape(sb * k, 3)
    if cfeat:
        x = jnp.concatenate([x, g_ref[:, 3:3 + cfeat]], axis=1)
    for li in range(nlayers):
        w = wrefs[2 * li][:, :]
        bb = wrefs[2 * li + 1][:, :]
        x = jnp.maximum(
            jnp.dot(x, w, preferred_element_type=jnp.float32) + bb, 0.0)
    cout = x.shape[1]
    o_ref[:, :] = jnp.max(x.reshape(sb, k, cout), axis=1)


def _mlp_call(gathered, centers_rows, layers, k, cfeat, sb):
    rows, d = gathered.shape
    bs = rows // k
    cout = layers[-1][0].shape[1]
    nlayers = len(layers)
    wargs = []
    in_specs = [
        pl.BlockSpec((sb * k, d), lambda g: (g, 0)),
        pl.BlockSpec((sb, 3), lambda g: (g, 0)),
    ]
    for wt, b2 in layers:
        wargs += [wt, b2]
        in_specs += [
            pl.BlockSpec(wt.shape, lambda g: (0, 0)),
            pl.BlockSpec(b2.shape, lambda g: (0, 0)),
        ]
    return pl.pallas_call(
        functools.partial(_mlp_kernel, k=k, cfeat=cfeat, sb=sb,
                          nlayers=nlayers),
        grid=(bs // sb,),
        in_specs=in_specs,
        out_specs=pl.BlockSpec((sb, cout), lambda g: (g, 0)),
        out_shape=jax.ShapeDtypeStruct((bs, cout), jnp.float32),
        interpret=False,
    )(gathered, centers_rows, *wargs)


_SC_CHUNK = 128


def _sc_gather_body(table_hbm, idx_hbm, out_hbm, idx_v, rows_v, sem, *,
                    m_per, nc):
    wid = lax.axis_index("s") * nc + lax.axis_index("c")
    base = pl.multiple_of(wid * m_per, 8)
    pltpu.sync_copy(idx_hbm.at[pl.ds(base, m_per)], idx_v)

    def body(i, _):
        off = pl.multiple_of(i * _SC_CHUNK, 8)
        pltpu.async_copy(
            table_hbm.at[idx_v.at[pl.ds(off, _SC_CHUNK)]],
            rows_v.at[pl.ds(off, _SC_CHUNK)], sem).wait()
        return 0

    lax.fori_loop(0, m_per // _SC_CHUNK, body, 0)
    pltpu.sync_copy(rows_v, out_hbm.at[pl.ds(base, m_per)])


def _gather_rows(table, idx_flat):
    m = idx_flat.shape[0]
    d = table.shape[1]
    info = plsc.get_sparse_core_info()
    nw = info.num_cores * info.num_subcores
    m_per = m // nw
    mesh = plsc.VectorSubcoreMesh(core_axis_name="c", subcore_axis_name="s")
    f = pl.kernel(
        functools.partial(_sc_gather_body, m_per=m_per, nc=info.num_cores),
        mesh=mesh,
        out_type=jax.ShapeDtypeStruct((m, d), jnp.float32),
        scratch_types=[
            pltpu.VMEM((m_per,), jnp.int32),
            pltpu.VMEM((m_per, d), jnp.float32),
            pltpu.SemaphoreType.DMA,
        ],
        compiler_params=pltpu.CompilerParams(use_tc_tiling_on_sc=False),
    )
    return f(table, idx_flat)


_SB_BQ = [256, 512, 256]
_SB_MLP = [512, 512, 256]
_PAD_D = [16, 48, 80]


def kernel(clouds, params):
    xyz = clouds[..., 0:3]
    feats = None
    for si, ((npoint, radius, nsample), stage_p) in enumerate(
            zip(_STAGES, params)):
        b, n, _ = xyz.shape
        xyz_t = jnp.transpose(xyz, (2, 0, 1))
        cent = _fps_call(xyz_t, npoint)
        new_xyz = jnp.transpose(cent, (1, 0, 2))
        idx = _bq_call(jnp.transpose(xyz, (0, 2, 1)), new_xyz, radius,
                       nsample, _SB_BQ[si])
        cfeat = 0 if feats is None else feats.shape[-1]
        base = xyz if feats is None else jnp.concatenate([xyz, feats], -1)
        d = _PAD_D[si]
        table = jnp.pad(base, ((0, 0), (0, 0), (0, d - 3 - cfeat)))
        table = table.reshape(b * n, d)
        gathered = _gather_rows(table, idx.reshape(-1))
        layers = []
        for lp in stage_p:
            s = lp["gamma"] * _BN_INV
            wt2 = (lp["W"] * s[:, None]).T
            b2 = (lp["b"] * s + lp["beta"])[None, :]
            layers.append((wt2, b2))
        fr = _mlp_call(gathered, new_xyz.reshape(b * npoint, 3), layers,
                       nsample, cfeat, _SB_MLP[si])
        feats = fr.reshape(b, npoint, fr.shape[-1])
        xyz = new_xyz
    return jnp.transpose(feats, (0, 2, 1))


# speedup vs baseline: 20.3518x; 1.0005x over previous
import functools

import jax
import jax.numpy as jnp
from jax import lax
from jax.experimental import pallas as pl
from jax.experimental.pallas import tpu as pltpu
from jax.experimental.pallas import tpu_sc as plsc

_B = 4
_N = 8192
_STAGES = [(1024, 0.1, 32), (512, 0.2, 32), (256, 0.4, 16)]
_BN_INV = 1.0 / (1.0 + 1e-5) ** 0.5


def _fps_kernel(x_ref, o_ref, *, npoint, n, b):
    x = x_ref[0, :, :]
    y = x_ref[1, :, :]
    z = x_ref[2, :, :]
    iota = lax.broadcasted_iota(jnp.int32, (b, n), 1)
    lx = x[:, 0:1]
    ly = y[:, 0:1]
    lz = z[:, 0:1]
    o_ref[0, :, :] = jnp.concatenate([lx, ly, lz], axis=1)
    dists = jnp.full((b, n), 1e10, dtype=jnp.float32)

    def body(i, carry):
        dists, lx, ly, lz = carry
        d = ((x - lx) ** 2 + (y - ly) ** 2) + (z - lz) ** 2
        dists = jnp.minimum(dists, d)
        nxt = jnp.argmax(dists, axis=1, keepdims=True).astype(jnp.int32)
        pick = iota == nxt
        lx = jnp.sum(jnp.where(pick, x, 0.0), axis=1, keepdims=True)
        ly = jnp.sum(jnp.where(pick, y, 0.0), axis=1, keepdims=True)
        lz = jnp.sum(jnp.where(pick, z, 0.0), axis=1, keepdims=True)
        o_ref[pl.ds(i, 1), :, :] = jnp.concatenate([lx, ly, lz], axis=1)[None]
        return dists, lx, ly, lz

    lax.fori_loop(1, npoint, body, (dists, lx, ly, lz))


def _fps_call(xyz_t, npoint):
    _, b, n = xyz_t.shape
    return pl.pallas_call(
        functools.partial(_fps_kernel, npoint=npoint, n=n, b=b),
        out_shape=jax.ShapeDtypeStruct((npoint, b, 3), jnp.float32),
        interpret=False,
    )(xyz_t)


def _bq_kernel(x_ref, c_ref, o_ref, *, n, k, r2, sb):
    x = x_ref[0, 0:1, :]
    y = x_ref[0, 1:2, :]
    z = x_ref[0, 2:3, :]
    cx = c_ref[0, :, 0:1]
    cy = c_ref[0, :, 1:2]
    cz = c_ref[0, :, 2:3]
    d2 = ((cx - x) ** 2 + (cy - y) ** 2) + (cz - z) ** 2
    iota = lax.broadcasted_iota(jnp.int32, (sb, n), 1)
    scores = jnp.where(d2 <= r2, iota, n)

    def cond(st):
        j, _, _, more = st
        return jnp.logical_and(j < k, more)

    kiota = lax.broadcasted_iota(jnp.int32, (sb, k), 1)

    def body(st):
        j, scores, out, _ = st
        m = jnp.min(scores, axis=1, keepdims=True)
        out = jnp.where(kiota == j, m, out)
        scores = jnp.where(scores == m, n, scores)
        return j + 1, scores, out, jnp.min(m) < n

    init = (jnp.int32(0), scores,
            jnp.full((sb, k), n, dtype=jnp.int32), True)
    _, _, idx, _ = lax.while_loop(cond, body, init)
    first = idx[:, 0:1]
    first = jnp.where(first < n, first, 0)
    idx = jnp.where(idx < n, idx, first)
    bi = pl.program_id(0)
    o_ref[0, :, :] = idx + bi * n


def _bq_call(xyz_t, centers, radius, k, sb):
    b, _, n = xyz_t.shape
    s = centers.shape[1]
    r2 = float(radius * radius)
    return pl.pallas_call(
        functools.partial(_bq_kernel, n=n, k=k, r2=r2, sb=sb),
        grid=(b, s // sb),
        in_specs=[
            pl.BlockSpec((1, 3, n), lambda bi, j: (bi, 0, 0)),
            pl.BlockSpec((1, sb, 3), lambda bi, j: (bi, j, 0)),
        ],
        out_specs=pl.BlockSpec((1, sb, k), lambda bi, j: (bi, j, 0)),
        out_shape=jax.ShapeDtypeStruct((b, s, k), jnp.int32),
        interpret=False,
    )(xyz_t, centers)


def _mlp_kernel(g_ref, c_ref, *refs, k, cfeat, sb, nlayers):
    o_ref = refs[-1]
    wrefs = refs[:-1]
    gx = g_ref[:, 0:3]
    c = c_ref[:, :]
    xyz = gx.reshape(sb, k, 3) - c[:, None, :]
    x = xyz.reshape(sb * k, 3)
    if cfeat:
        x = jnp.concatenate([x, g_ref[:, 3:3 + cfeat]], axis=1)
    for li in range(nlayers):
        w = wrefs[2 * li][:, :]
        bb = wrefs[2 * li + 1][:, :]
        x = jnp.maximum(
            jnp.dot(x, w, preferred_element_type=jnp.float32) + bb, 0.0)
    cout = x.shape[1]
    o_ref[:, :] = jnp.max(x.reshape(sb, k, cout), axis=1)


def _mlp_call(gathered, centers_rows, layers, k, cfeat, sb):
    rows, d = gathered.shape
    bs = rows // k
    cout = layers[-1][0].shape[1]
    nlayers = len(layers)
    wargs = []
    in_specs = [
        pl.BlockSpec((sb * k, d), lambda g: (g, 0)),
        pl.BlockSpec((sb, 3), lambda g: (g, 0)),
    ]
    for wt, b2 in layers:
        wargs += [wt, b2]
        in_specs += [
            pl.BlockSpec(wt.shape, lambda g: (0, 0)),
            pl.BlockSpec(b2.shape, lambda g: (0, 0)),
        ]
    return pl.pallas_call(
        functools.partial(_mlp_kernel, k=k, cfeat=cfeat, sb=sb,
                          nlayers=nlayers),
        grid=(bs // sb,),
        in_specs=in_specs,
        out_specs=pl.BlockSpec((sb, cout), lambda g: (g, 0)),
        out_shape=jax.ShapeDtypeStruct((bs, cout), jnp.float32),
        interpret=False,
    )(gathered, centers_rows, *wargs)


_SC_CHUNK = 128


def _sc_gather_body(table_hbm, idx_hbm, out_hbm, idx_v, rows_v, sem, *,
                    m_per, nc):
    wid = lax.axis_index("s") * nc + lax.axis_index("c")
    base = pl.multiple_of(wid * m_per, 8)
    pltpu.sync_copy(idx_hbm.at[pl.ds(base, m_per)], idx_v)

    def body(i, _):
        off = pl.multiple_of(i * _SC_CHUNK, 8)
        pltpu.async_copy(
            table_hbm.at[idx_v.at[pl.ds(off, _SC_CHUNK)]],
            rows_v.at[pl.ds(off, _SC_CHUNK)], sem).wait()
        return 0

    lax.fori_loop(0, m_per // _SC_CHUNK, body, 0)
    pltpu.sync_copy(rows_v, out_hbm.at[pl.ds(base, m_per)])


def _gather_rows(table, idx_flat):
    m = idx_flat.shape[0]
    d = table.shape[1]
    info = plsc.get_sparse_core_info()
    nw = info.num_cores * info.num_subcores
    m_per = m // nw
    mesh = plsc.VectorSubcoreMesh(core_axis_name="c", subcore_axis_name="s")
    f = pl.kernel(
        functools.partial(_sc_gather_body, m_per=m_per, nc=info.num_cores),
        mesh=mesh,
        out_type=jax.ShapeDtypeStruct((m, d), jnp.float32),
        scratch_types=[
            pltpu.VMEM((m_per,), jnp.int32),
            pltpu.VMEM((m_per, d), jnp.float32),
            pltpu.SemaphoreType.DMA,
        ],
        compiler_params=pltpu.CompilerParams(use_tc_tiling_on_sc=False),
    )
    return f(table, idx_flat)


_SB_BQ = [256, 512, 256]
_SB_MLP = [512, 512, 256]
_PAD_D = [16, 48, 80]


def kernel(clouds, params):
    b = clouds.shape[0]
    xyz = clouds[..., 0:3]

    xyzs, idxs = [], []
    cur = xyz
    for si, (npoint, radius, nsample) in enumerate(_STAGES):
        xyz_t = jnp.transpose(cur, (2, 0, 1))
        cent = _fps_call(xyz_t, npoint)
        new_xyz = jnp.transpose(cent, (1, 0, 2))
        idx = _bq_call(jnp.transpose(cur, (0, 2, 1)), new_xyz, radius,
                       nsample, _SB_BQ[si])
        xyzs.append(cur)
        idxs.append(idx)
        cur = new_xyz

    feats = None
    for si, ((npoint, radius, nsample), stage_p) in enumerate(
            zip(_STAGES, params)):
        xyz_s = xyzs[si]
        n = xyz_s.shape[1]
        cfeat = 0 if feats is None else feats.shape[-1]
        base = xyz_s if feats is None else jnp.concatenate([xyz_s, feats], -1)
        d = _PAD_D[si]
        table = jnp.pad(base, ((0, 0), (0, 0), (0, d - 3 - cfeat)))
        table = table.reshape(b * n, d)
        gathered = _gather_rows(table, idxs[si].reshape(-1))
        layers = []
        for lp in stage_p:
            s = lp["gamma"] * _BN_INV
            wt2 = (lp["W"] * s[:, None]).T
            b2 = (lp["b"] * s + lp["beta"])[None, :]
            layers.append((wt2, b2))
        new_xyz = xyzs[si + 1] if si + 1 < len(xyzs) else cur
        fr = _mlp_call(gathered, new_xyz.reshape(b * npoint, 3), layers,
                       nsample, cfeat, _SB_MLP[si])
        feats = fr.reshape(b, npoint, fr.shape[-1])
    return jnp.transpose(feats, (0, 2, 1))


# speedup vs baseline: 20.3866x; 1.0017x over previous
import functools

import jax
import jax.numpy as jnp
from jax import lax
from jax.experimental import pallas as pl
from jax.experimental.pallas import tpu as pltpu
from jax.experimental.pallas import tpu_sc as plsc

_B = 4
_N = 8192
_STAGES = [(1024, 0.1, 32), (512, 0.2, 32), (256, 0.4, 16)]
_BN_INV = 1.0 / (1.0 + 1e-5) ** 0.5


def _fps_kernel(x_ref, o_ref, *, npoint, n, b):
    x = x_ref[0, :, :]
    y = x_ref[1, :, :]
    z = x_ref[2, :, :]
    iota = lax.broadcasted_iota(jnp.int32, (b, n), 1)
    lx = x[:, 0:1]
    ly = y[:, 0:1]
    lz = z[:, 0:1]
    o_ref[0, :, :] = jnp.concatenate([lx, ly, lz], axis=1)
    dists = jnp.full((b, n), 1e10, dtype=jnp.float32)

    def body(i, carry):
        dists, lx, ly, lz = carry
        d = ((x - lx) ** 2 + (y - ly) ** 2) + (z - lz) ** 2
        dists = jnp.minimum(dists, d)
        nxt = jnp.argmax(dists, axis=1, keepdims=True).astype(jnp.int32)
        pick = iota == nxt
        lx = jnp.sum(jnp.where(pick, x, 0.0), axis=1, keepdims=True)
        ly = jnp.sum(jnp.where(pick, y, 0.0), axis=1, keepdims=True)
        lz = jnp.sum(jnp.where(pick, z, 0.0), axis=1, keepdims=True)
        o_ref[pl.ds(i, 1), :, :] = jnp.concatenate([lx, ly, lz], axis=1)[None]
        return dists, lx, ly, lz

    lax.fori_loop(1, npoint, body, (dists, lx, ly, lz))


def _fps_call(xyz_t, npoint):
    _, b, n = xyz_t.shape
    return pl.pallas_call(
        functools.partial(_fps_kernel, npoint=npoint, n=n, b=b),
        out_shape=jax.ShapeDtypeStruct((npoint, b, 3), jnp.float32),
        interpret=False,
    )(xyz_t)


def _bq_kernel(x_ref, c_ref, o_ref, *, n, k, r2, sb):
    x = x_ref[0, 0:1, :]
    y = x_ref[0, 1:2, :]
    z = x_ref[0, 2:3, :]
    cx = c_ref[0, :, 0:1]
    cy = c_ref[0, :, 1:2]
    cz = c_ref[0, :, 2:3]
    d2 = ((cx - x) ** 2 + (cy - y) ** 2) + (cz - z) ** 2
    iota = lax.broadcasted_iota(jnp.int32, (sb, n), 1)
    scores = jnp.where(d2 <= r2, iota, n)

    def cond(st):
        j, _, _, more = st
        return jnp.logical_and(j < k, more)

    kiota = lax.broadcasted_iota(jnp.int32, (sb, k), 1)

    def body(st):
        j, scores, out, _ = st
        m = jnp.min(scores, axis=1, keepdims=True)
        out = jnp.where(kiota == j, m, out)
        scores = jnp.where(scores == m, n, scores)
        return j + 1, scores, out, jnp.min(m) < n

    init = (jnp.int32(0), scores,
            jnp.full((sb, k), n, dtype=jnp.int32), True)
    _, _, idx, _ = lax.while_loop(cond, body, init)
    first = idx[:, 0:1]
    first = jnp.where(first < n, first, 0)
    idx = jnp.where(idx < n, idx, first)
    bi = pl.program_id(0)
    o_ref[0, :, :] = idx + bi * n


def _bq_call(xyz_t, centers, radius, k, sb):
    b, _, n = xyz_t.shape
    s = centers.shape[1]
    r2 = float(radius * radius)
    return pl.pallas_call(
        functools.partial(_bq_kernel, n=n, k=k, r2=r2, sb=sb),
        grid=(b, s // sb),
        in_specs=[
            pl.BlockSpec((1, 3, n), lambda bi, j: (bi, 0, 0)),
            pl.BlockSpec((1, sb, 3), lambda bi, j: (bi, j, 0)),
        ],
        out_specs=pl.BlockSpec((1, sb, k), lambda bi, j: (bi, j, 0)),
        out_shape=jax.ShapeDtypeStruct((b, s, k), jnp.int32),
        interpret=False,
    )(xyz_t, centers)


def _mlp_kernel(g_ref, c_ref, *refs, k, cfeat, sb, nlayers):
    o_ref = refs[-1]
    wrefs = refs[:-1]
    gx = g_ref[:, 0:3]
    c = c_ref[:, :]
    xyz = gx.reshape(sb, k, 3) - c[:, None, :]
    x = xyz.reshape(sb * k, 3)
    if cfeat:
        x = jnp.concatenate([x, g_ref[:, 3:3 + cfeat]], axis=1)
    for li in range(nlayers):
        w = wrefs[2 * li][:, :]
        bb = wrefs[2 * li + 1][:, :]
        x = jnp.maximum(
            jnp.dot(x, w, preferred_element_type=jnp.float32) + bb, 0.0)
    cout = x.shape[1]
    o_ref[:, :] = jnp.max(x.reshape(sb, k, cout), axis=1)


def _mlp_call(gathered, centers_rows, layers, k, cfeat, sb):
    rows, d = gathered.shape
    bs = rows // k
    cout = layers[-1][0].shape[1]
    nlayers = len(layers)
    wargs = []
    in_specs = [
        pl.BlockSpec((sb * k, d), lambda g: (g, 0)),
        pl.BlockSpec((sb, 3), lambda g: (g, 0)),
    ]
    for wt, b2 in layers:
        wargs += [wt, b2]
        in_specs += [
            pl.BlockSpec(wt.shape, lambda g: (0, 0)),
            pl.BlockSpec(b2.shape, lambda g: (0, 0)),
        ]
    return pl.pallas_call(
        functools.partial(_mlp_kernel, k=k, cfeat=cfeat, sb=sb,
                          nlayers=nlayers),
        grid=(bs // sb,),
        in_specs=in_specs,
        out_specs=pl.BlockSpec((sb, cout), lambda g: (g, 0)),
        out_shape=jax.ShapeDtypeStruct((bs, cout), jnp.float32),
        interpret=False,
    )(gathered, centers_rows, *wargs)


_SC_CHUNK = 128


def _sc_gather_body(table_hbm, idx_hbm, out_hbm, idx_v, rows_v, sem, *,
                    m_per, nc):
    wid = lax.axis_index("s") * nc + lax.axis_index("c")
    base = pl.multiple_of(wid * m_per, 8)
    pltpu.sync_copy(idx_hbm.at[pl.ds(base, m_per)], idx_v)
    nch = m_per // _SC_CHUNK

    def _copy(i):
        off = pl.multiple_of(i * _SC_CHUNK, 8)
        return pltpu.make_async_copy(
            table_hbm.at[idx_v.at[pl.ds(off, _SC_CHUNK)]],
            rows_v.at[pl.ds(off, _SC_CHUNK)], sem)

    _copy(0).start()

    def body(i, _):
        _copy(i + 1).start()
        _copy(i).wait()
        return 0

    lax.fori_loop(0, nch - 1, body, 0)
    _copy(nch - 1).wait()
    pltpu.sync_copy(rows_v, out_hbm.at[pl.ds(base, m_per)])


def _gather_rows(table, idx_flat):
    m = idx_flat.shape[0]
    d = table.shape[1]
    info = plsc.get_sparse_core_info()
    nw = info.num_cores * info.num_subcores
    m_per = m // nw
    mesh = plsc.VectorSubcoreMesh(core_axis_name="c", subcore_axis_name="s")
    f = pl.kernel(
        functools.partial(_sc_gather_body, m_per=m_per, nc=info.num_cores),
        mesh=mesh,
        out_type=jax.ShapeDtypeStruct((m, d), jnp.float32),
        scratch_types=[
            pltpu.VMEM((m_per,), jnp.int32),
            pltpu.VMEM((m_per, d), jnp.float32),
            pltpu.SemaphoreType.DMA,
        ],
        compiler_params=pltpu.CompilerParams(use_tc_tiling_on_sc=False),
    )
    return f(table, idx_flat)


_SB_BQ = [256, 512, 256]
_SB_MLP = [512, 512, 256]
_PAD_D = [16, 48, 80]


def kernel(clouds, params):
    b = clouds.shape[0]
    xyz = clouds[..., 0:3]

    xyzs, idxs = [], []
    cur = xyz
    for si, (npoint, radius, nsample) in enumerate(_STAGES):
        xyz_t = jnp.transpose(cur, (2, 0, 1))
        cent = _fps_call(xyz_t, npoint)
        new_xyz = jnp.transpose(cent, (1, 0, 2))
        idx = _bq_call(jnp.transpose(cur, (0, 2, 1)), new_xyz, radius,
                       nsample, _SB_BQ[si])
        xyzs.append(cur)
        idxs.append(idx)
        cur = new_xyz

    feats = None
    for si, ((npoint, radius, nsample), stage_p) in enumerate(
            zip(_STAGES, params)):
        xyz_s = xyzs[si]
        n = xyz_s.shape[1]
        cfeat = 0 if feats is None else feats.shape[-1]
        base = xyz_s if feats is None else jnp.concatenate([xyz_s, feats], -1)
        d = _PAD_D[si]
        table = jnp.pad(base, ((0, 0), (0, 0), (0, d - 3 - cfeat)))
        table = table.reshape(b * n, d)
        gathered = _gather_rows(table, idxs[si].reshape(-1))
        layers = []
        for lp in stage_p:
            s = lp["gamma"] * _BN_INV
            wt2 = (lp["W"] * s[:, None]).T
            b2 = (lp["b"] * s + lp["beta"])[None, :]
            layers.append((wt2, b2))
        new_xyz = xyzs[si + 1] if si + 1 < len(xyzs) else cur
        fr = _mlp_call(gathered, new_xyz.reshape(b * npoint, 3), layers,
                       nsample, cfeat, _SB_MLP[si])
        feats = fr.reshape(b, npoint, fr.shape[-1])
    return jnp.transpose(feats, (0, 2, 1))
